# Initial kernel scaffold; baseline (speedup 1.0000x reference)
import jax
import jax.numpy as jnp
from jax.experimental import pallas as pl


def kernel(x, table):
    raise NotImplementedError("write your pallas kernel here")


# speedup vs baseline: 3.2722x; 3.2722x over previous
import functools
import math

import jax
import jax.numpy as jnp
from jax import lax
from jax.experimental import pallas as pl
from jax.experimental.pallas import tpu as pltpu
from jax.experimental.pallas import tpu_sc as plsc

D_MODEL = 64
SCALE = math.sqrt(D_MODEL)

NUM_CORES = 2
NUM_SUBCORES = 16
NUM_WORKERS = NUM_CORES * NUM_SUBCORES

GROUP = 128


def _scale_table_body(t_ref, o_ref):
    o_ref[...] = t_ref[...] * SCALE


def _scale_table(table):
    vocab, d = table.shape
    rows_per_block = 2000
    grid = vocab // rows_per_block
    return pl.pallas_call(
        _scale_table_body,
        out_shape=jax.ShapeDtypeStruct((vocab, d), jnp.float32),
        grid=(grid,),
        in_specs=[pl.BlockSpec((rows_per_block, d), lambda i: (i, 0))],
        out_specs=pl.BlockSpec((rows_per_block, d), lambda i: (i, 0)),
    )(table)


@functools.cache
def _make_gather(batch, vocab, d):
    assert batch % (NUM_WORKERS * GROUP) == 0
    b_per_w = batch // NUM_WORKERS
    n_groups = b_per_w // GROUP
    mesh = plsc.VectorSubcoreMesh(
        core_axis_name="c",
        subcore_axis_name="s",
        num_cores=NUM_CORES,
        num_subcores=NUM_SUBCORES,
    )

    @functools.partial(
        pl.kernel,
        out_type=jax.ShapeDtypeStruct((batch, d), jnp.float32),
        mesh=mesh,
        scratch_types=[
            pltpu.VMEM((b_per_w,), jnp.int32),
            pltpu.VMEM((GROUP, d), jnp.float32),
            pltpu.SemaphoreType.DMA,
        ],
        compiler_params=pltpu.CompilerParams(use_tc_tiling_on_sc=False),
    )
    def gather_kernel(table_hbm, idx_hbm, out_hbm, idx_v, rows_v, sem):
        wid = lax.axis_index("s") * NUM_CORES + lax.axis_index("c")
        base = wid * b_per_w
        pltpu.sync_copy(idx_hbm.at[pl.ds(base, b_per_w)], idx_v)

        def body(g, carry):
            off = g * GROUP
            pltpu.async_copy(
                table_hbm.at[idx_v.at[pl.ds(off, GROUP)]], rows_v, sem
            ).wait()
            pltpu.sync_copy(rows_v, out_hbm.at[pl.ds(base + off, GROUP)])
            return carry

        lax.fori_loop(0, n_groups, body, 0)

    return gather_kernel


def kernel(x, table):
    b, t = x.shape
    vocab, d = table.shape
    scaled = _scale_table(table)
    idx = x.reshape(b * t).astype(jnp.int32)
    out = _make_gather(b * t, vocab, d)(scaled, idx)
    return out.reshape(b, t, d)


# speedup vs baseline: 3.4720x; 1.0611x over previous
import functools
import math

import jax
import jax.numpy as jnp
from jax import lax
from jax.experimental import pallas as pl
from jax.experimental.pallas import tpu as pltpu
from jax.experimental.pallas import tpu_sc as plsc

D_MODEL = 64
SCALE = math.sqrt(D_MODEL)

NUM_CORES = 2
NUM_SUBCORES = 16
NUM_WORKERS = NUM_CORES * NUM_SUBCORES

GROUP = 128


def _scale_table_body(t_ref, o_ref):
    o_ref[...] = t_ref[...] * SCALE


def _scale_table(table):
    vocab, d = table.shape
    rows_per_block = 2000
    grid = vocab // rows_per_block
    return pl.pallas_call(
        _scale_table_body,
        out_shape=jax.ShapeDtypeStruct((vocab, d), jnp.float32),
        grid=(grid,),
        in_specs=[pl.BlockSpec((rows_per_block, d), lambda i: (i, 0))],
        out_specs=pl.BlockSpec((rows_per_block, d), lambda i: (i, 0)),
    )(table)


@functools.cache
def _make_gather(nb, nt, vocab, d):
    assert nb % NUM_WORKERS == 0
    b_per_w = nb // NUM_WORKERS
    n_idx = b_per_w * nt
    g0 = min(nt, GROUP)
    g1 = nt - g0
    mesh = plsc.VectorSubcoreMesh(
        core_axis_name="c",
        subcore_axis_name="s",
        num_cores=NUM_CORES,
        num_subcores=NUM_SUBCORES,
    )

    @functools.partial(
        pl.kernel,
        out_type=jax.ShapeDtypeStruct((nb, nt, d), jnp.float32),
        mesh=mesh,
        scratch_types=[
            pltpu.VMEM((n_idx,), jnp.int32),
            pltpu.VMEM((nt, d), jnp.float32),
            pltpu.SemaphoreType.DMA,
        ],
        compiler_params=pltpu.CompilerParams(use_tc_tiling_on_sc=False),
    )
    def gather_kernel(table_hbm, idx_hbm, out_hbm, idx_v, rows_v, sem):
        wid = lax.axis_index("s") * NUM_CORES + lax.axis_index("c")
        b_base = wid * b_per_w
        pltpu.sync_copy(idx_hbm.at[pl.ds(b_base * nt, n_idx)], idx_v)

        def body(j, carry):
            off = j * nt
            cp0 = pltpu.async_copy(
                table_hbm.at[idx_v.at[pl.ds(off, g0)]],
                rows_v.at[pl.ds(0, g0)],
                sem,
            )
            cp1 = pltpu.async_copy(
                table_hbm.at[idx_v.at[pl.ds(off + g0, g1)]],
                rows_v.at[pl.ds(g0, g1)],
                sem,
            )
            cp0.wait()
            cp1.wait()
            pltpu.sync_copy(rows_v, out_hbm.at[b_base + j])
            return carry

        lax.fori_loop(0, b_per_w, body, 0)

    return gather_kernel


def kernel(x, table):
    b, t = x.shape
    vocab, d = table.shape
    scaled = _scale_table(table)
    idx = x.reshape(b * t).astype(jnp.int32)
    return _make_gather(b, t, vocab, d)(scaled, idx)


# speedup vs baseline: 5.3984x; 1.5548x over previous
import functools
import math

import jax
import jax.numpy as jnp
from jax import lax
from jax.experimental import pallas as pl
from jax.experimental.pallas import tpu as pltpu
from jax.experimental.pallas import tpu_sc as plsc

D_MODEL = 64
SCALE = math.sqrt(D_MODEL)

NUM_CORES = 2
NUM_SUBCORES = 16
NUM_WORKERS = NUM_CORES * NUM_SUBCORES

GROUP = 128


def _scale_table_body(t_ref, o_ref):
    o_ref[...] = t_ref[...] * SCALE


def _scale_table(table):
    vocab, d = table.shape
    rows_per_block = 2000
    grid = vocab // rows_per_block
    return pl.pallas_call(
        _scale_table_body,
        out_shape=jax.ShapeDtypeStruct((vocab, d), jnp.float32),
        grid=(grid,),
        in_specs=[pl.BlockSpec((rows_per_block, d), lambda i: (i, 0))],
        out_specs=pl.BlockSpec((rows_per_block, d), lambda i: (i, 0)),
    )(table)


@functools.cache
def _make_gather(nb, nt, vocab, d):
    assert nb % NUM_WORKERS == 0
    b_per_w = nb // NUM_WORKERS
    n_idx = b_per_w * nt
    g0 = min(nt, GROUP)
    g1 = nt - g0
    mesh = plsc.VectorSubcoreMesh(
        core_axis_name="c",
        subcore_axis_name="s",
        num_cores=NUM_CORES,
        num_subcores=NUM_SUBCORES,
    )

    @functools.partial(
        pl.kernel,
        out_type=jax.ShapeDtypeStruct((nb, nt, 2 * d), jnp.float32),
        mesh=mesh,
        scratch_types=[
            pltpu.VMEM((n_idx,), jnp.int32),
            pltpu.VMEM((nt, d), jnp.float32),
            pltpu.SemaphoreType.DMA,
        ],
        compiler_params=pltpu.CompilerParams(use_tc_tiling_on_sc=False),
    )
    def gather_kernel(table_hbm, idx_hbm, out_hbm, idx_v, rows_v, sem):
        wid = lax.axis_index("s") * NUM_CORES + lax.axis_index("c")
        b_base = wid * b_per_w
        pltpu.sync_copy(idx_hbm.at[pl.ds(b_base * nt, n_idx)], idx_v)

        def body(j, carry):
            off = j * nt
            cp0 = pltpu.async_copy(
                table_hbm.at[idx_v.at[pl.ds(off, g0)]],
                rows_v.at[pl.ds(0, g0)],
                sem,
            )
            cp1 = pltpu.async_copy(
                table_hbm.at[idx_v.at[pl.ds(off + g0, g1)]],
                rows_v.at[pl.ds(g0, g1)],
                sem,
            )
            cp0.wait()
            cp1.wait()
            pltpu.sync_copy(rows_v, out_hbm.at[b_base + j, :, pl.ds(0, d)])
            return carry

        lax.fori_loop(0, b_per_w, body, 0)

    return gather_kernel


def kernel(x, table):
    b, t = x.shape
    vocab, d = table.shape
    scaled = _scale_table(table)
    idx = x.reshape(b * t).astype(jnp.int32)
    out2 = _make_gather(b, t, vocab, d)(scaled, idx)
    return lax.slice(out2, (0, 0, 0), (b, t, d))
